# baseline (speedup 1.0000x reference)
import functools

import jax
import jax.numpy as jnp
from jax import lax
from jax.experimental import pallas as pl
from jax.experimental.pallas import tpu as pltpu
from jax.experimental.pallas import tpu_sc as plsc

VOCAB = 100000
D = 64
B, S = 1024, 200
NW = 32
TOK_PER_W = B * S // NW
SEQ_PER_W = TOK_PER_W // S
HALF = 100
ROWS_PER_W = TOK_PER_W // 2

_mesh = plsc.VectorSubcoreMesh(core_axis_name="c", subcore_axis_name="s")


@functools.partial(
    pl.kernel,
    mesh=_mesh,
    out_type=jax.ShapeDtypeStruct((B * S // 2, 2 * D), jnp.float32),
    scratch_types=[
        pltpu.VMEM((SEQ_PER_W * 2, HALF), jnp.int32),
        pltpu.VMEM((TOK_PER_W + 16,), jnp.int32),
        pltpu.VMEM((HALF, 2 * D), jnp.float32),
        pltpu.VMEM((HALF, D), jnp.float32),
        pltpu.VMEM((HALF, D), jnp.float32),
        pltpu.VMEM((HALF, 2 * D), jnp.float32),
        pltpu.SemaphoreType.DMA,
    ],
    compiler_params=pltpu.CompilerParams(use_tc_tiling_on_sc=False),
)
def _encoder(ids_stream, ids_chk, table, pos2, out, idx_v, chk_v, pos_v, buf_a, buf_b, stg_v, sem):
    w = lax.axis_index("s") * 2 + lax.axis_index("c")
    pltpu.sync_copy(ids_stream.at[w], idx_v)
    pltpu.sync_copy(ids_chk.at[w], chk_v.at[pl.ds(0, TOK_PER_W)])
    pltpu.sync_copy(pos2, pos_v)

    def _mn(i, acc):
        return jnp.minimum(acc, chk_v[pl.ds(i * 16, 16)])

    acc = lax.fori_loop(0, TOK_PER_W // 16, _mn,
                        jnp.full((16,), jnp.iinfo(jnp.int32).max, jnp.int32))
    lanes = lax.iota(jnp.int32, 16)
    for shift in (8, 4, 2, 1):
        perm = lax.rem(lanes + shift, 16)
        g = lax.gather(
            acc, perm[:, None],
            dimension_numbers=lax.GatherDimensionNumbers(
                offset_dims=(), collapsed_slice_dims=(0,), start_index_map=(0,)),
            slice_sizes=(1,), mode=lax.GatherScatterMode.PROMISE_IN_BOUNDS)
        acc = jnp.minimum(acc, g)
    has_pad = acc[0] == 0

    def _chunk(cidx, carry):
        base = cidx * S
        cp1 = pltpu.async_copy(table.at[idx_v.at[2 * cidx]], buf_a, sem)
        cp2 = pltpu.async_copy(table.at[idx_v.at[2 * cidx + 1]], buf_b, sem)
        cp1.wait()
        cp2.wait()

        @pl.when(has_pad)
        def _():
            def _fix(j, c2):
                idv = chk_v[pl.ds(base + j, 16)]
                @pl.when(idv[0] == 0)
                def _():
                    zero = jnp.zeros((16,), jnp.float32)
                    row = lax.shift_right_logical(j, 1)
                    even = lax.rem(j, 2) == 0
                    for k in range(4):
                        @pl.when(even)
                        def _():
                            buf_a[row, pl.ds(k * 16, 16)] = zero
                        @pl.when(jnp.logical_not(even))
                        def _():
                            buf_b[row, pl.ds(k * 16, 16)] = zero
                return c2
            lax.fori_loop(0, S, _fix, 0)

        def _add(r, c2):
            for k in range(4):
                sl = pl.ds(k * 16, 16)
                sr = pl.ds(D + k * 16, 16)
                stg_v[r, sl] = buf_a[r, sl] + pos_v[r, sl]
                stg_v[r, sr] = buf_b[r, sl] + pos_v[r, sr]
            return c2
        lax.fori_loop(0, HALF, _add, 0)

        pltpu.sync_copy(stg_v, out.at[pl.ds(w * ROWS_PER_W + cidx * HALF, HALF)])
        return carry

    lax.fori_loop(0, SEQ_PER_W, _chunk, 0)


def kernel(input_ids, token_table, pos_table):
    ids = input_ids.astype(jnp.int32)
    ids_stream = (ids.reshape(NW, SEQ_PER_W, HALF, 2)
                  .transpose(0, 1, 3, 2)
                  .reshape(NW, SEQ_PER_W * 2, HALF))
    ids_chk = ids.reshape(NW, TOK_PER_W)
    pos2 = pos_table[:S].reshape(HALF, 2 * D)
    out = _encoder(ids_stream, ids_chk, token_table, pos2)
    return out.reshape(B, S, D)


# speedup vs baseline: 1.4021x; 1.4021x over previous
import functools

import jax
import jax.numpy as jnp
from jax import lax
from jax.experimental import pallas as pl
from jax.experimental.pallas import tpu as pltpu
from jax.experimental.pallas import tpu_sc as plsc

VOCAB = 100000
D = 64
B, S = 1024, 200
NW = 32
TOK_PER_W = B * S // NW
SEQ_PER_W = TOK_PER_W // S
HALF = 100
NBUF = 4

_mesh = plsc.VectorSubcoreMesh(core_axis_name="c", subcore_axis_name="s")


@functools.partial(
    pl.kernel,
    mesh=_mesh,
    out_type=jax.ShapeDtypeStruct((B * S, D), jnp.float32),
    scratch_types=[
        pltpu.VMEM((SEQ_PER_W * 2, HALF), jnp.int32),
        pltpu.VMEM((TOK_PER_W + 16,), jnp.int32),
        pltpu.VMEM((S, D), jnp.float32),
        pltpu.VMEM((S, D), jnp.float32),
        pltpu.VMEM((S, D), jnp.float32),
        pltpu.VMEM((S, D), jnp.float32),
        pltpu.VMEM((S, D), jnp.float32),
        pltpu.SemaphoreType.DMA,
        pltpu.SemaphoreType.DMA,
        pltpu.SemaphoreType.DMA,
        pltpu.SemaphoreType.DMA,
        pltpu.SemaphoreType.DMA,
        pltpu.SemaphoreType.DMA,
        pltpu.SemaphoreType.DMA,
        pltpu.SemaphoreType.DMA,
    ],
    compiler_params=pltpu.CompilerParams(use_tc_tiling_on_sc=False),
)
def _encoder(ids_stream, ids_chk, table, pos, out,
             idx_v, chk_v, pos_v, b0, b1, b2, b3,
             g0, g1, g2, g3, s0, s1, s2, s3):
    bufs = (b0, b1, b2, b3)
    gsems = (g0, g1, g2, g3)
    ssems = (s0, s1, s2, s3)

    w = lax.axis_index("s") * 2 + lax.axis_index("c")
    pltpu.sync_copy(ids_stream.at[w], idx_v)
    pltpu.sync_copy(ids_chk.at[w], chk_v.at[pl.ds(0, TOK_PER_W)])
    pltpu.sync_copy(pos.at[pl.ds(0, S)], pos_v)

    def gathers(slot, c):
        return (pltpu.make_async_copy(table.at[idx_v.at[2 * c]],
                                      bufs[slot].at[pl.ds(0, HALF)],
                                      gsems[slot]),
                pltpu.make_async_copy(table.at[idx_v.at[2 * c + 1]],
                                      bufs[slot].at[pl.ds(HALF, HALF)],
                                      gsems[slot]))

    def store(slot, c):
        return pltpu.make_async_copy(
            bufs[slot], out.at[pl.ds(w * TOK_PER_W + c * S, S)], ssems[slot])

    def _mn(i, acc):
        return jnp.minimum(acc, chk_v[pl.ds(i * 16, 16)])

    acc = lax.fori_loop(0, TOK_PER_W // 16, _mn,
                        jnp.full((16,), jnp.iinfo(jnp.int32).max, jnp.int32))
    lanes = lax.iota(jnp.int32, 16)
    for shift in (8, 4, 2, 1):
        g = lax.gather(
            acc, lax.rem(lanes + shift, 16)[:, None],
            dimension_numbers=lax.GatherDimensionNumbers(
                offset_dims=(), collapsed_slice_dims=(0,), start_index_map=(0,)),
            slice_sizes=(1,), mode=lax.GatherScatterMode.PROMISE_IN_BOUNDS)
        acc = jnp.minimum(acc, g)
    has_pad = acc[0] == 0

    for c0 in range(NBUF - 1):
        a, b = gathers(c0, jnp.int32(c0))
        a.start()
        b.start()

    def compute(slot, c):
        buf = bufs[slot]
        base = c * S

        @pl.when(has_pad)
        def _():
            def _fix(r, c2):
                idv = chk_v[pl.ds(base + r, 16)]
                @pl.when(idv[0] == 0)
                def _():
                    zero = jnp.zeros((16,), jnp.float32)
                    for k in range(4):
                        buf[r, pl.ds(k * 16, 16)] = zero
                return c2
            lax.fori_loop(0, S, _fix, 0)

        def _add(r, c2):
            for k in range(4):
                sl = pl.ds(k * 16, 16)
                buf[r, sl] = buf[r, sl] + pos_v[r, sl]
            return c2
        lax.fori_loop(0, S, _add, 0)

    def _iter(i, carry):
        for j in range(NBUF):
            c = NBUF * i + j
            nxt = c + NBUF - 1
            tgt = (j + NBUF - 1) % NBUF

            def _prefetch():
                a, b = gathers(tgt, nxt)
                a.start()
                b.start()

            if j == 0:
                @pl.when(i > 0)
                def _():
                    store(tgt, nxt - NBUF).wait()
                _prefetch()
            else:
                @pl.when(i < SEQ_PER_W // NBUF - 1)
                def _():
                    store(tgt, nxt - NBUF).wait()
                    _prefetch()

            ga, gb = gathers(j, c)
            ga.wait()
            gb.wait()
            compute(j, c)
            store(j, c).start()
        return carry

    lax.fori_loop(0, SEQ_PER_W // NBUF, _iter, 0)

    for j in range(NBUF):
        store(j, jnp.int32(SEQ_PER_W - NBUF + j)).wait()


def kernel(input_ids, token_table, pos_table):
    ids = input_ids.astype(jnp.int32)
    ids_stream = ids.reshape(NW, SEQ_PER_W * 2, HALF)
    ids_chk = ids.reshape(NW, TOK_PER_W)
    out = _encoder(ids_stream, ids_chk, token_table, pos_table)
    return out.reshape(B, S, D)
